# Initial kernel scaffold; baseline (speedup 1.0000x reference)
import jax
import jax.numpy as jnp
from jax.experimental import pallas as pl


def kernel(text_token, cache, W):
    raise NotImplementedError("write your pallas kernel here")


# speedup vs baseline: 1.9325x; 1.9325x over previous
import jax
import jax.numpy as jnp
from jax.experimental import pallas as pl

ALPHA = 0.2
BLOCK = 2048


def _fused_body(x_ref, cache_ref, a_ref, b_ref, o_ref):
    x = x_ref[...]
    cache = cache_ref[...]
    n = jnp.sqrt(jnp.sum(x * x, axis=1, keepdims=True))
    xn = x / jnp.maximum(n, 1e-12)
    s = jax.lax.dot_general(
        xn, cache, (((1,), (1,)), ((), ())), preferred_element_type=jnp.float32
    )
    m = jnp.max(s, axis=1, keepdims=True)
    e = jnp.exp(s - m)
    p = e / jnp.sum(e, axis=1, keepdims=True)
    fine = jnp.dot(p, cache, preferred_element_type=jnp.float32)
    o_ref[...] = (
        jnp.dot(x, a_ref[...], preferred_element_type=jnp.float32)
        + jnp.dot(fine, b_ref[...], preferred_element_type=jnp.float32)
    )


def kernel(text_token, cache, W):
    n_rows, d = text_token.shape
    m = cache.shape[0]
    a = ALPHA * W[:, :d].T + jnp.eye(d, dtype=W.dtype)
    b = ALPHA * W[:, d:].T
    out = pl.pallas_call(
        _fused_body,
        grid=(n_rows // BLOCK,),
        in_specs=[
            pl.BlockSpec((BLOCK, d), lambda i: (i, 0)),
            pl.BlockSpec((m, d), lambda i: (0, 0)),
            pl.BlockSpec((d, d), lambda i: (0, 0)),
            pl.BlockSpec((d, d), lambda i: (0, 0)),
        ],
        out_specs=pl.BlockSpec((BLOCK, d), lambda i: (i, 0)),
        out_shape=jax.ShapeDtypeStruct((n_rows, d), text_token.dtype),
    )(text_token, cache, a, b)
    return (out, 0.0)
